# baseline (speedup 1.0000x reference)
import functools

import jax
import jax.numpy as jnp
from jax import lax
from jax.experimental import pallas as pl
from jax.experimental.pallas import tpu as pltpu
from jax.experimental.pallas import tpu_sc as plsc

_H = 768
_I = 256
_E = 64
_K = 2
_SCALE = 1.0
_BT = 128
_TMAX = 96
_TM = 256


def _shared_router_body(x_ref, wg_ref, wu_ref, wd_ref, wr_ref,
                        base_ref, probs_ref):
    xt = x_ref[...]
    g = jnp.dot(xt, wg_ref[...], preferred_element_type=jnp.float32)
    u = jnp.dot(xt, wu_ref[...], preferred_element_type=jnp.float32)
    h = jax.nn.sigmoid(g) * u
    so = jnp.dot(h, wd_ref[...], preferred_element_type=jnp.float32)
    base_ref[...] = xt + so
    logits = jnp.dot(xt, wr_ref[...], preferred_element_type=jnp.float32)
    m = jnp.max(logits, axis=-1, keepdims=True)
    e = jnp.exp(logits - m)
    probs_ref[...] = e / jnp.sum(e, axis=-1, keepdims=True)


def _shared_router(x2d, Wg_s, Wu_s, Wd_s, Wr):
    s = x2d.shape[0]
    return pl.pallas_call(
        _shared_router_body,
        grid=(s // _TM,),
        in_specs=[
            pl.BlockSpec((_TM, _H), lambda i: (i, 0)),
            pl.BlockSpec((_H, _I), lambda i: (0, 0)),
            pl.BlockSpec((_H, _I), lambda i: (0, 0)),
            pl.BlockSpec((_I, _H), lambda i: (0, 0)),
            pl.BlockSpec((_H, _E), lambda i: (0, 0)),
        ],
        out_specs=[
            pl.BlockSpec((_TM, _H), lambda i: (i, 0)),
            pl.BlockSpec((_TM, _E), lambda i: (i, 0)),
        ],
        out_shape=[
            jax.ShapeDtypeStruct((s, _H), jnp.float32),
            jax.ShapeDtypeStruct((s, _E), jnp.float32),
        ],
    )(x2d, Wg_s, Wu_s, Wd_s, Wr)


def _group_mlp_body(te_ref, xs_ref, wg_ref, wu_ref, wd_ref, rw_ref, out_ref):
    del te_ref
    xt = xs_ref[...]
    g = jnp.dot(xt, wg_ref[0], preferred_element_type=jnp.float32)
    u = jnp.dot(xt, wu_ref[0], preferred_element_type=jnp.float32)
    h = jax.nn.sigmoid(g) * u
    o = jnp.dot(h, wd_ref[0], preferred_element_type=jnp.float32)
    out_ref[...] = o * rw_ref[0, 0, :][:, None]


def _group_mlp(xs, Wg, Wu, Wd, row_w, tile_eid):
    grid_spec = pltpu.PrefetchScalarGridSpec(
        num_scalar_prefetch=1,
        grid=(_TMAX,),
        in_specs=[
            pl.BlockSpec((_BT, _H), lambda t, te: (t, 0)),
            pl.BlockSpec((1, _H, _I), lambda t, te: (te[t], 0, 0)),
            pl.BlockSpec((1, _H, _I), lambda t, te: (te[t], 0, 0)),
            pl.BlockSpec((1, _I, _H), lambda t, te: (te[t], 0, 0)),
            pl.BlockSpec((1, 1, _BT), lambda t, te: (t, 0, 0)),
        ],
        out_specs=pl.BlockSpec((_BT, _H), lambda t, te: (t, 0)),
    )
    return pl.pallas_call(
        _group_mlp_body,
        grid_spec=grid_spec,
        out_shape=jax.ShapeDtypeStruct((_TMAX * _BT, _H), jnp.float32),
    )(tile_eid, xs, Wg, Wu, Wd, row_w.reshape(_TMAX, 1, _BT))


def _sc_gather_rows(table, idx, chunk):
    b = idx.shape[0]
    d = table.shape[1]
    nw = 32
    b_per_w = b // nw
    mesh = plsc.VectorSubcoreMesh(core_axis_name="c", subcore_axis_name="s",
                                  num_cores=2, num_subcores=16)

    @functools.partial(
        pl.kernel,
        out_type=jax.ShapeDtypeStruct((b, d), jnp.float32),
        mesh=mesh,
        scratch_types=[
            pltpu.VMEM((chunk,), jnp.int32),
            pltpu.VMEM((chunk, d), jnp.float32),
            pltpu.SemaphoreType.DMA,
        ],
    )
    def k(table_hbm, idx_hbm, out_hbm, idx_v, rows_v, sem):
        wid = lax.axis_index("s") * 2 + lax.axis_index("c")
        base = wid * b_per_w

        @pl.loop(0, b_per_w, step=chunk)
        def _(off):
            pltpu.sync_copy(idx_hbm.at[pl.ds(base + off, chunk)], idx_v)
            pltpu.async_copy(table_hbm.at[idx_v], rows_v, sem).wait()
            pltpu.sync_copy(rows_v, out_hbm.at[pl.ds(base + off, chunk)])

    return k(table, idx)


def _plan(probs):
    s = probs.shape[0]
    n = s * _K
    top_w, top_idx = lax.top_k(probs, _K)
    aw = (top_w * _SCALE).reshape(n)
    eid = top_idx.reshape(n).astype(jnp.int32)
    tok = jnp.arange(n, dtype=jnp.int32) // _K
    oh = jax.nn.one_hot(eid, _E, dtype=jnp.int32)
    counts = jnp.sum(oh, axis=0)
    rank = jnp.take_along_axis(jnp.cumsum(oh, axis=0) - oh,
                               eid[:, None], axis=1)[:, 0]
    tiles_e = (counts + _BT - 1) // _BT
    pad_base = (jnp.cumsum(tiles_e) - tiles_e) * _BT
    dest = pad_base[eid] + rank
    src_tok = jnp.zeros((_TMAX * _BT,), jnp.int32).at[dest].set(tok)
    row_w = jnp.zeros((_TMAX * _BT,), jnp.float32).at[dest].set(aw)
    tile_eid = jnp.zeros((_TMAX,), jnp.int32).at[dest // _BT].set(eid)
    tile_eid = lax.cummax(tile_eid)
    return src_tok, row_w, tile_eid, dest


def kernel(x, Wg_s, Wu_s, Wd_s, Wg, Wu, Wd, Wr):
    b, s, h = x.shape
    flat = x.reshape(s, h)
    base, probs = _shared_router(flat, Wg_s, Wu_s, Wd_s, Wr)
    src_tok, row_w, tile_eid, dest = _plan(probs)
    xs = _sc_gather_rows(flat, src_tok, 64)
    ys = _group_mlp(xs, Wg, Wu, Wd, row_w, tile_eid)
    picked = _sc_gather_rows(ys, dest, 64)
    routed = picked.reshape(s, _K, h).sum(axis=1)
    return (base + routed).reshape(b, s, h)


# speedup vs baseline: 1.9502x; 1.9502x over previous
import functools

import jax
import jax.numpy as jnp
from jax import lax
from jax.experimental import pallas as pl
from jax.experimental.pallas import tpu as pltpu
from jax.experimental.pallas import tpu_sc as plsc

_H = 768
_I = 256
_E = 64
_K = 2
_SCALE = 1.0
_BT = 128
_TMAX = 96
_TM = 256


def _shared_router_body(x_ref, wg_ref, wu_ref, wd_ref, wr_ref,
                        base_ref, probs_ref):
    xt = x_ref[...]
    g = jnp.dot(xt, wg_ref[...], preferred_element_type=jnp.float32)
    u = jnp.dot(xt, wu_ref[...], preferred_element_type=jnp.float32)
    h = jax.nn.sigmoid(g) * u
    so = jnp.dot(h, wd_ref[...], preferred_element_type=jnp.float32)
    base_ref[...] = xt + so
    logits = jnp.dot(xt, wr_ref[...], preferred_element_type=jnp.float32)
    m = jnp.max(logits, axis=-1, keepdims=True)
    e = jnp.exp(logits - m)
    probs_ref[...] = e / jnp.sum(e, axis=-1, keepdims=True)


def _shared_router(x2d, Wg_s, Wu_s, Wd_s, Wr):
    s = x2d.shape[0]
    return pl.pallas_call(
        _shared_router_body,
        grid=(s // _TM,),
        in_specs=[
            pl.BlockSpec((_TM, _H), lambda i: (i, 0)),
            pl.BlockSpec((_H, _I), lambda i: (0, 0)),
            pl.BlockSpec((_H, _I), lambda i: (0, 0)),
            pl.BlockSpec((_I, _H), lambda i: (0, 0)),
            pl.BlockSpec((_H, _E), lambda i: (0, 0)),
        ],
        out_specs=[
            pl.BlockSpec((_TM, _H), lambda i: (i, 0)),
            pl.BlockSpec((_TM, _E), lambda i: (i, 0)),
        ],
        out_shape=[
            jax.ShapeDtypeStruct((s, _H), jnp.float32),
            jax.ShapeDtypeStruct((s, _E), jnp.float32),
        ],
    )(x2d, Wg_s, Wu_s, Wd_s, Wr)


def _group_mlp_body(te_ref, xs_ref, wg_ref, wu_ref, wd_ref, rw_ref, out_ref):
    del te_ref
    xt = xs_ref[...]
    g = jnp.dot(xt, wg_ref[0], preferred_element_type=jnp.float32)
    u = jnp.dot(xt, wu_ref[0], preferred_element_type=jnp.float32)
    h = jax.nn.sigmoid(g) * u
    o = jnp.dot(h, wd_ref[0], preferred_element_type=jnp.float32)
    out_ref[...] = o * rw_ref[0, 0, :][:, None]


def _group_mlp(xs, Wg, Wu, Wd, row_w, tile_eid):
    grid_spec = pltpu.PrefetchScalarGridSpec(
        num_scalar_prefetch=1,
        grid=(_TMAX,),
        in_specs=[
            pl.BlockSpec((_BT, _H), lambda t, te: (t, 0)),
            pl.BlockSpec((1, _H, _I), lambda t, te: (te[t], 0, 0)),
            pl.BlockSpec((1, _H, _I), lambda t, te: (te[t], 0, 0)),
            pl.BlockSpec((1, _I, _H), lambda t, te: (te[t], 0, 0)),
            pl.BlockSpec((1, 1, _BT), lambda t, te: (t, 0, 0)),
        ],
        out_specs=pl.BlockSpec((_BT, _H), lambda t, te: (t, 0)),
    )
    return pl.pallas_call(
        _group_mlp_body,
        grid_spec=grid_spec,
        out_shape=jax.ShapeDtypeStruct((_TMAX * _BT, _H), jnp.float32),
    )(tile_eid, xs, Wg, Wu, Wd, row_w.reshape(_TMAX, 1, _BT))


def _sc_gather_rows(table, idx, chunk):
    b = idx.shape[0]
    d = table.shape[1]
    nw = 32
    b_per_w = b // nw
    mesh = plsc.VectorSubcoreMesh(core_axis_name="c", subcore_axis_name="s",
                                  num_cores=2, num_subcores=16)

    @functools.partial(
        pl.kernel,
        out_type=jax.ShapeDtypeStruct((b, d), jnp.float32),
        mesh=mesh,
        scratch_types=[
            pltpu.VMEM((chunk,), jnp.int32),
            pltpu.VMEM((chunk, d), jnp.float32),
            pltpu.SemaphoreType.DMA,
        ],
    )
    def k(table_hbm, idx_hbm, out_hbm, idx_v, rows_v, sem):
        wid = lax.axis_index("s") * 2 + lax.axis_index("c")
        base = wid * b_per_w

        @pl.loop(0, b_per_w, step=chunk)
        def _(off):
            pltpu.sync_copy(idx_hbm.at[pl.ds(base + off, chunk)], idx_v)
            pltpu.async_copy(table_hbm.at[idx_v], rows_v, sem).wait()
            pltpu.sync_copy(rows_v, out_hbm.at[pl.ds(base + off, chunk)])

    return k(table, idx)


def _plan(probs):
    s = probs.shape[0]
    n = s * _K
    top_w, top_idx = lax.top_k(probs, _K)
    aw = (top_w * _SCALE).reshape(n)
    eid = top_idx.reshape(n).astype(jnp.int32)
    tok = jnp.arange(n, dtype=jnp.int32) // _K
    oh = jax.nn.one_hot(eid, _E, dtype=jnp.int32)
    counts = jnp.sum(oh, axis=0)
    rank = jnp.take_along_axis(jnp.cumsum(oh, axis=0) - oh,
                               eid[:, None], axis=1)[:, 0]
    tiles_e = (counts + _BT - 1) // _BT
    pad_base = (jnp.cumsum(tiles_e) - tiles_e) * _BT
    dest = pad_base[eid] + rank
    pad_idx = jnp.arange(_TMAX * _BT, dtype=jnp.int32) % s
    src_tok = pad_idx.at[dest].set(tok)
    row_w = jnp.zeros((_TMAX * _BT,), jnp.float32).at[dest].set(aw)
    tile_eid = jnp.zeros((_TMAX,), jnp.int32).at[dest // _BT].set(eid)
    tile_eid = lax.cummax(tile_eid)
    return src_tok, row_w, tile_eid, dest


def kernel(x, Wg_s, Wu_s, Wd_s, Wg, Wu, Wd, Wr):
    b, s, h = x.shape
    flat = x.reshape(s, h)
    base, probs = _shared_router(flat, Wg_s, Wu_s, Wd_s, Wr)
    src_tok, row_w, tile_eid, dest = _plan(probs)
    xs = _sc_gather_rows(flat, src_tok, 64)
    ys = _group_mlp(xs, Wg, Wu, Wd, row_w, tile_eid)
    picked = _sc_gather_rows(ys, dest, 64)
    routed = picked.reshape(s, _K, h).sum(axis=1)
    return (base + routed).reshape(b, s, h)


# speedup vs baseline: 2.4703x; 1.2667x over previous
import functools

import jax
import jax.numpy as jnp
from jax import lax
from jax.experimental import pallas as pl
from jax.experimental.pallas import tpu as pltpu
from jax.experimental.pallas import tpu_sc as plsc

_H = 768
_I = 256
_E = 64
_K = 2
_SCALE = 1.0
_BT = 128
_TMAX = 96
_TM = 256


def _shared_router_body(x_ref, wg_ref, wu_ref, wd_ref, wr_ref,
                        base_ref, probs_ref):
    xt = x_ref[...]
    g = jnp.dot(xt, wg_ref[...], preferred_element_type=jnp.float32)
    u = jnp.dot(xt, wu_ref[...], preferred_element_type=jnp.float32)
    h = jax.nn.sigmoid(g) * u
    so = jnp.dot(h, wd_ref[...], preferred_element_type=jnp.float32)
    base_ref[...] = xt + so
    logits = jnp.dot(xt, wr_ref[...], preferred_element_type=jnp.float32)
    m = jnp.max(logits, axis=-1, keepdims=True)
    e = jnp.exp(logits - m)
    probs_ref[...] = e / jnp.sum(e, axis=-1, keepdims=True)


def _shared_router(x2d, Wg_s, Wu_s, Wd_s, Wr):
    s = x2d.shape[0]
    return pl.pallas_call(
        _shared_router_body,
        grid=(s // _TM,),
        in_specs=[
            pl.BlockSpec((_TM, _H), lambda i: (i, 0)),
            pl.BlockSpec((_H, _I), lambda i: (0, 0)),
            pl.BlockSpec((_H, _I), lambda i: (0, 0)),
            pl.BlockSpec((_I, _H), lambda i: (0, 0)),
            pl.BlockSpec((_H, _E), lambda i: (0, 0)),
        ],
        out_specs=[
            pl.BlockSpec((_TM, _H), lambda i: (i, 0)),
            pl.BlockSpec((_TM, _E), lambda i: (i, 0)),
        ],
        out_shape=[
            jax.ShapeDtypeStruct((s, _H), jnp.float32),
            jax.ShapeDtypeStruct((s, _E), jnp.float32),
        ],
    )(x2d, Wg_s, Wu_s, Wd_s, Wr)


def _group_mlp_body(te_ref, tv_ref, xs_ref, wg_ref, wu_ref, wd_ref, out_ref):
    del te_ref
    t = pl.program_id(0)

    @pl.when(tv_ref[t] == 1)
    def _():
        xt = xs_ref[...]
        g = jnp.dot(xt, wg_ref[0], preferred_element_type=jnp.float32)
        u = jnp.dot(xt, wu_ref[0], preferred_element_type=jnp.float32)
        h = jax.nn.sigmoid(g) * u
        out_ref[...] = jnp.dot(h, wd_ref[0], preferred_element_type=jnp.float32)


def _group_mlp(xs, Wg, Wu, Wd, tile_eid, tile_valid):
    grid_spec = pltpu.PrefetchScalarGridSpec(
        num_scalar_prefetch=2,
        grid=(_TMAX,),
        in_specs=[
            pl.BlockSpec((_BT, _H), lambda t, te, tv: (t, 0)),
            pl.BlockSpec((1, _H, _I), lambda t, te, tv: (te[t], 0, 0)),
            pl.BlockSpec((1, _H, _I), lambda t, te, tv: (te[t], 0, 0)),
            pl.BlockSpec((1, _I, _H), lambda t, te, tv: (te[t], 0, 0)),
        ],
        out_specs=pl.BlockSpec((_BT, _H), lambda t, te, tv: (t, 0)),
    )
    return pl.pallas_call(
        _group_mlp_body,
        grid_spec=grid_spec,
        out_shape=jax.ShapeDtypeStruct((_TMAX * _BT, _H), jnp.float32),
    )(tile_eid, tile_valid, xs, Wg, Wu, Wd)


def _sc_gather_rows(table, idx, chunk):
    b = idx.shape[0]
    d = table.shape[1]
    nw = 32
    b_per_w = b // nw
    mesh = plsc.VectorSubcoreMesh(core_axis_name="c", subcore_axis_name="s",
                                  num_cores=2, num_subcores=16)

    @functools.partial(
        pl.kernel,
        out_type=jax.ShapeDtypeStruct((b, d), jnp.float32),
        mesh=mesh,
        scratch_types=[
            pltpu.VMEM((chunk,), jnp.int32),
            pltpu.VMEM((chunk, d), jnp.float32),
            pltpu.SemaphoreType.DMA,
        ],
    )
    def k(table_hbm, idx_hbm, out_hbm, idx_v, rows_v, sem):
        wid = lax.axis_index("s") * 2 + lax.axis_index("c")
        base = wid * b_per_w

        @pl.loop(0, b_per_w, step=chunk)
        def _(off):
            pltpu.sync_copy(idx_hbm.at[pl.ds(base + off, chunk)], idx_v)
            pltpu.async_copy(table_hbm.at[idx_v], rows_v, sem).wait()
            pltpu.sync_copy(rows_v, out_hbm.at[pl.ds(base + off, chunk)])

    return k(table, idx)


def _plan(probs):
    s = probs.shape[0]
    n = s * _K
    top_w, top_idx = lax.top_k(probs, _K)
    eid = top_idx.reshape(n).astype(jnp.int32)
    tok = jnp.arange(n, dtype=jnp.int32) // _K
    oh = jax.nn.one_hot(eid, _E, dtype=jnp.int32)
    counts = jnp.sum(oh, axis=0)
    rank = jnp.take_along_axis(jnp.cumsum(oh, axis=0) - oh,
                               eid[:, None], axis=1)[:, 0]
    tiles_e = (counts + _BT - 1) // _BT
    tile_start = jnp.cumsum(tiles_e) - tiles_e
    dest = tile_start[eid] * _BT + rank
    t_ar = jnp.arange(_TMAX, dtype=jnp.int32)
    tile_eid = jnp.sum((tile_start[None, :] <= t_ar[:, None]).astype(jnp.int32),
                       axis=1) - 1
    tile_valid = (t_ar < jnp.sum(tiles_e)).astype(jnp.int32)
    pad_idx = jnp.arange(_TMAX * _BT, dtype=jnp.int32) % s
    src_tok = pad_idx.at[dest].set(tok)
    dest_km = dest.reshape(s, _K).T.reshape(n)
    return src_tok, tile_eid, tile_valid, dest_km, top_w


def kernel(x, Wg_s, Wu_s, Wd_s, Wg, Wu, Wd, Wr):
    b, s, h = x.shape
    flat = x.reshape(s, h)
    base, probs = _shared_router(flat, Wg_s, Wu_s, Wd_s, Wr)
    src_tok, tile_eid, tile_valid, dest_km, top_w = _plan(probs)
    xs = _sc_gather_rows(flat, src_tok, 64)
    ys = _group_mlp(xs, Wg, Wu, Wd, tile_eid, tile_valid)
    picked = _sc_gather_rows(ys, dest_km, 64)
    w = top_w * _SCALE
    out = base + w[:, 0:1] * picked[:s] + w[:, 1:2] * picked[s:]
    return out.reshape(b, s, h)


# speedup vs baseline: 2.9843x; 1.2080x over previous
import functools

import jax
import jax.numpy as jnp
from jax import lax
from jax.experimental import pallas as pl
from jax.experimental.pallas import tpu as pltpu
from jax.experimental.pallas import tpu_sc as plsc

_H = 768
_I = 256
_E = 64
_K = 2
_SCALE = 1.0
_BT = 128
_TMAX = 96
_TM = 256


def _shared_router_body(x_ref, wg_ref, wu_ref, wd_ref, wr_ref,
                        base_ref, probs_ref):
    xt = x_ref[...]
    g = jnp.dot(xt, wg_ref[...], preferred_element_type=jnp.float32)
    u = jnp.dot(xt, wu_ref[...], preferred_element_type=jnp.float32)
    h = jax.nn.sigmoid(g) * u
    so = jnp.dot(h, wd_ref[...], preferred_element_type=jnp.float32)
    base_ref[...] = xt + so
    logits = jnp.dot(xt, wr_ref[...], preferred_element_type=jnp.float32)
    m = jnp.max(logits, axis=-1, keepdims=True)
    e = jnp.exp(logits - m)
    probs_ref[...] = e / jnp.sum(e, axis=-1, keepdims=True)


def _shared_router(x2d, Wg_s, Wu_s, Wd_s, Wr):
    s = x2d.shape[0]
    return pl.pallas_call(
        _shared_router_body,
        grid=(s // _TM,),
        in_specs=[
            pl.BlockSpec((_TM, _H), lambda i: (i, 0)),
            pl.BlockSpec((_H, _I), lambda i: (0, 0)),
            pl.BlockSpec((_H, _I), lambda i: (0, 0)),
            pl.BlockSpec((_I, _H), lambda i: (0, 0)),
            pl.BlockSpec((_H, _E), lambda i: (0, 0)),
        ],
        out_specs=[
            pl.BlockSpec((_TM, _H), lambda i: (i, 0)),
            pl.BlockSpec((_TM, _E), lambda i: (i, 0)),
        ],
        out_shape=[
            jax.ShapeDtypeStruct((s, _H), jnp.float32),
            jax.ShapeDtypeStruct((s, _E), jnp.float32),
        ],
    )(x2d, Wg_s, Wu_s, Wd_s, Wr)


def _group_mlp_body(te_ref, tv_ref, xs_ref, wg_ref, wu_ref, wd_ref, out_ref):
    del te_ref
    t = pl.program_id(0)

    @pl.when(tv_ref[t] == 1)
    def _():
        xt = xs_ref[...].astype(jnp.bfloat16)
        g = jnp.dot(xt, wg_ref[0].astype(jnp.bfloat16),
                    preferred_element_type=jnp.float32)
        u = jnp.dot(xt, wu_ref[0].astype(jnp.bfloat16),
                    preferred_element_type=jnp.float32)
        h = (jax.nn.sigmoid(g) * u).astype(jnp.bfloat16)
        out_ref[...] = jnp.dot(h, wd_ref[0].astype(jnp.bfloat16),
                               preferred_element_type=jnp.float32)


def _group_mlp(xs, Wg, Wu, Wd, tile_eid, tile_valid):
    grid_spec = pltpu.PrefetchScalarGridSpec(
        num_scalar_prefetch=2,
        grid=(_TMAX,),
        in_specs=[
            pl.BlockSpec((_BT, _H), lambda t, te, tv: (t, 0)),
            pl.BlockSpec((1, _H, _I), lambda t, te, tv: (te[t], 0, 0)),
            pl.BlockSpec((1, _H, _I), lambda t, te, tv: (te[t], 0, 0)),
            pl.BlockSpec((1, _I, _H), lambda t, te, tv: (te[t], 0, 0)),
        ],
        out_specs=pl.BlockSpec((_BT, _H), lambda t, te, tv: (t, 0)),
    )
    return pl.pallas_call(
        _group_mlp_body,
        grid_spec=grid_spec,
        out_shape=jax.ShapeDtypeStruct((_TMAX * _BT, _H), jnp.float32),
    )(tile_eid, tile_valid, xs, Wg, Wu, Wd)


def _sc_gather_rows(table, idx, chunk):
    b = idx.shape[0]
    d = table.shape[1]
    nw = 32
    b_per_w = b // nw
    mesh = plsc.VectorSubcoreMesh(core_axis_name="c", subcore_axis_name="s",
                                  num_cores=2, num_subcores=16)

    @functools.partial(
        pl.kernel,
        out_type=jax.ShapeDtypeStruct((b, d), jnp.float32),
        mesh=mesh,
        scratch_types=[
            pltpu.VMEM((chunk,), jnp.int32),
            pltpu.VMEM((chunk, d), jnp.float32),
            pltpu.SemaphoreType.DMA,
        ],
    )
    def k(table_hbm, idx_hbm, out_hbm, idx_v, rows_v, sem):
        wid = lax.axis_index("s") * 2 + lax.axis_index("c")
        base = wid * b_per_w

        @pl.loop(0, b_per_w, step=chunk)
        def _(off):
            pltpu.sync_copy(idx_hbm.at[pl.ds(base + off, chunk)], idx_v)
            pltpu.async_copy(table_hbm.at[idx_v], rows_v, sem).wait()
            pltpu.sync_copy(rows_v, out_hbm.at[pl.ds(base + off, chunk)])

    return k(table, idx)


def _combine_body(base_ref, p0_ref, p1_ref, w_ref, out_ref):
    w0 = w_ref[0, 0, :][:, None]
    w1 = w_ref[1, 0, :][:, None]
    out_ref[...] = base_ref[...] + w0 * p0_ref[...] + w1 * p1_ref[...]


def _combine(base, picked, w2s):
    s = base.shape[0]
    return pl.pallas_call(
        _combine_body,
        grid=(s // _TM,),
        in_specs=[
            pl.BlockSpec((_TM, _H), lambda i: (i, 0)),
            pl.BlockSpec((_TM, _H), lambda i: (i, 0)),
            pl.BlockSpec((_TM, _H), lambda i, _o=s // _TM: (_o + i, 0)),
            pl.BlockSpec((_K, 1, _TM), lambda i: (0, 0, i)),
        ],
        out_specs=pl.BlockSpec((_TM, _H), lambda i: (i, 0)),
        out_shape=jax.ShapeDtypeStruct((s, _H), jnp.float32),
    )(base, picked, picked, w2s.reshape(_K, 1, s))


def _plan(probs):
    s = probs.shape[0]
    n = s * _K
    top_w, top_idx = lax.top_k(probs, _K)
    eid = jnp.concatenate([top_idx[:, 0], top_idx[:, 1]]).astype(jnp.int32)
    tok = jnp.arange(n, dtype=jnp.int32) % s
    ohf = (eid[:, None] == jnp.arange(_E, dtype=jnp.int32)[None, :])
    ohf = ohf.astype(jnp.float32)
    cum = jnp.cumsum(ohf, axis=0)
    counts = cum[-1].astype(jnp.int32)
    tiles_e = (counts + _BT - 1) // _BT
    tile_start = jnp.cumsum(tiles_e) - tiles_e
    base_f = (tile_start * _BT).astype(jnp.float32)
    dest = jnp.sum(ohf * (base_f[None, :] + cum - 1.0),
                   axis=1).astype(jnp.int32)
    t_ar = jnp.arange(_TMAX, dtype=jnp.int32)
    tile_eid = jnp.sum((tile_start[None, :] <= t_ar[:, None]).astype(jnp.int32),
                       axis=1) - 1
    tile_valid = (t_ar < jnp.sum(tiles_e)).astype(jnp.int32)
    pad_idx = jnp.arange(_TMAX * _BT, dtype=jnp.int32) % s
    src_tok = pad_idx.at[dest].set(tok)
    w2s = jnp.concatenate([top_w[:, 0], top_w[:, 1]]) * _SCALE
    return src_tok, tile_eid, tile_valid, dest, w2s


def kernel(x, Wg_s, Wu_s, Wd_s, Wg, Wu, Wd, Wr):
    b, s, h = x.shape
    flat = x.reshape(s, h)
    base, probs = _shared_router(flat, Wg_s, Wu_s, Wd_s, Wr)
    src_tok, tile_eid, tile_valid, dest, w2s = _plan(probs)
    xs = _sc_gather_rows(flat, src_tok, 64)
    ys = _group_mlp(xs, Wg, Wu, Wd, tile_eid, tile_valid)
    picked = _sc_gather_rows(ys, dest, 64)
    return _combine(base, picked, w2s).reshape(b, s, h)


# speedup vs baseline: 3.5243x; 1.1810x over previous
import functools

import jax
import jax.numpy as jnp
from jax import lax
from jax.experimental import pallas as pl
from jax.experimental.pallas import tpu as pltpu
from jax.experimental.pallas import tpu_sc as plsc

_H = 768
_I = 256
_E = 64
_K = 2
_SCALE = 1.0
_BT = 128
_TMAX = 96
_TM = 256


def _shared_router_body(x_ref, wg_ref, wu_ref, wd_ref, wr_ref,
                        base_ref, probs_ref):
    xt = x_ref[...]
    g = jnp.dot(xt, wg_ref[...], preferred_element_type=jnp.float32)
    u = jnp.dot(xt, wu_ref[...], preferred_element_type=jnp.float32)
    h = jax.nn.sigmoid(g) * u
    so = jnp.dot(h, wd_ref[...], preferred_element_type=jnp.float32)
    base_ref[...] = xt + so
    logits = jnp.dot(xt, wr_ref[...], preferred_element_type=jnp.float32)
    m = jnp.max(logits, axis=-1, keepdims=True)
    e = jnp.exp(logits - m)
    probs_ref[...] = e / jnp.sum(e, axis=-1, keepdims=True)


def _shared_router(x2d, Wg_s, Wu_s, Wd_s, Wr):
    s = x2d.shape[0]
    return pl.pallas_call(
        _shared_router_body,
        grid=(s // _TM,),
        in_specs=[
            pl.BlockSpec((_TM, _H), lambda i: (i, 0)),
            pl.BlockSpec((_H, _I), lambda i: (0, 0)),
            pl.BlockSpec((_H, _I), lambda i: (0, 0)),
            pl.BlockSpec((_I, _H), lambda i: (0, 0)),
            pl.BlockSpec((_H, _E), lambda i: (0, 0)),
        ],
        out_specs=[
            pl.BlockSpec((_TM, _H), lambda i: (i, 0)),
            pl.BlockSpec((_TM, _E), lambda i: (i, 0)),
        ],
        out_shape=[
            jax.ShapeDtypeStruct((s, _H), jnp.float32),
            jax.ShapeDtypeStruct((s, _E), jnp.float32),
        ],
    )(x2d, Wg_s, Wu_s, Wd_s, Wr)


def _group_mlp_body(te_ref, tv_ref, xs_ref, wg_ref, wu_ref, wd_ref, out_ref):
    del te_ref
    t = pl.program_id(0)

    @pl.when(tv_ref[t] == 1)
    def _():
        xt = xs_ref[...].astype(jnp.bfloat16)
        g = jnp.dot(xt, wg_ref[0].astype(jnp.bfloat16),
                    preferred_element_type=jnp.float32)
        u = jnp.dot(xt, wu_ref[0].astype(jnp.bfloat16),
                    preferred_element_type=jnp.float32)
        h = (jax.nn.sigmoid(g) * u).astype(jnp.bfloat16)
        out_ref[...] = jnp.dot(h, wd_ref[0].astype(jnp.bfloat16),
                               preferred_element_type=jnp.float32)


def _group_mlp(xs, Wg, Wu, Wd, tile_eid, tile_valid):
    grid_spec = pltpu.PrefetchScalarGridSpec(
        num_scalar_prefetch=2,
        grid=(_TMAX,),
        in_specs=[
            pl.BlockSpec((_BT, _H), lambda t, te, tv: (t, 0)),
            pl.BlockSpec((1, _H, _I), lambda t, te, tv: (te[t], 0, 0)),
            pl.BlockSpec((1, _H, _I), lambda t, te, tv: (te[t], 0, 0)),
            pl.BlockSpec((1, _I, _H), lambda t, te, tv: (te[t], 0, 0)),
        ],
        out_specs=pl.BlockSpec((_BT, _H), lambda t, te, tv: (t, 0)),
    )
    return pl.pallas_call(
        _group_mlp_body,
        grid_spec=grid_spec,
        out_shape=jax.ShapeDtypeStruct((_TMAX * _BT, _H), jnp.float32),
    )(tile_eid, tile_valid, xs, Wg, Wu, Wd)


def _sc_gather_rows(table, idx, chunk):
    b = idx.shape[0]
    d = table.shape[1]
    nw = 32
    b_per_w = b // nw
    mesh = plsc.VectorSubcoreMesh(core_axis_name="c", subcore_axis_name="s",
                                  num_cores=2, num_subcores=16)

    @functools.partial(
        pl.kernel,
        out_type=jax.ShapeDtypeStruct((b, d), jnp.float32),
        mesh=mesh,
        scratch_types=[
            pltpu.VMEM((chunk,), jnp.int32),
            pltpu.VMEM((chunk, d), jnp.float32),
            pltpu.SemaphoreType.DMA,
        ],
    )
    def k(table_hbm, idx_hbm, out_hbm, idx_v, rows_v, sem):
        wid = lax.axis_index("s") * 2 + lax.axis_index("c")
        base = wid * b_per_w

        @pl.loop(0, b_per_w, step=chunk)
        def _(off):
            pltpu.sync_copy(idx_hbm.at[pl.ds(base + off, chunk)], idx_v)
            pltpu.async_copy(table_hbm.at[idx_v], rows_v, sem).wait()
            pltpu.sync_copy(rows_v, out_hbm.at[pl.ds(base + off, chunk)])

    return k(table, idx)


def _sc_scatter_rows(table, dst_idx, out_rows, chunk):
    s, d = table.shape
    n = dst_idx.shape[0]
    nw = 32
    b_per_w = n // nw
    mesh = plsc.VectorSubcoreMesh(core_axis_name="c", subcore_axis_name="s",
                                  num_cores=2, num_subcores=16)

    @functools.partial(
        pl.kernel,
        out_type=jax.ShapeDtypeStruct((out_rows, d), jnp.float32),
        mesh=mesh,
        scratch_types=[
            pltpu.VMEM((chunk,), jnp.int32),
            pltpu.VMEM((chunk, d), jnp.float32),
            pltpu.SemaphoreType.DMA,
        ],
    )
    def k(table_hbm, idx_hbm, out_hbm, idx_v, rows_v, sem):
        wid = lax.axis_index("s") * 2 + lax.axis_index("c")
        base = wid * b_per_w

        @pl.loop(0, b_per_w, step=chunk)
        def _(off):
            i0 = base + off
            src = lax.rem(i0, s)
            pltpu.sync_copy(idx_hbm.at[pl.ds(i0, chunk)], idx_v)
            pltpu.sync_copy(table_hbm.at[pl.ds(src, chunk)], rows_v)
            pltpu.async_copy(rows_v, out_hbm.at[idx_v], sem).wait()

    return k(table, dst_idx)


def _combine_body(base_ref, p0_ref, p1_ref, w_ref, out_ref):
    w0 = w_ref[0, 0, :][:, None]
    w1 = w_ref[1, 0, :][:, None]
    out_ref[...] = base_ref[...] + w0 * p0_ref[...] + w1 * p1_ref[...]


def _combine(base, picked, w2s):
    s = base.shape[0]
    return pl.pallas_call(
        _combine_body,
        grid=(s // _TM,),
        in_specs=[
            pl.BlockSpec((_TM, _H), lambda i: (i, 0)),
            pl.BlockSpec((_TM, _H), lambda i: (i, 0)),
            pl.BlockSpec((_TM, _H), lambda i, _o=s // _TM: (_o + i, 0)),
            pl.BlockSpec((_K, 1, _TM), lambda i: (0, 0, i)),
        ],
        out_specs=pl.BlockSpec((_TM, _H), lambda i: (i, 0)),
        out_shape=jax.ShapeDtypeStruct((s, _H), jnp.float32),
    )(base, picked, picked, w2s.reshape(_K, 1, s))


def _plan(probs):
    s = probs.shape[0]
    n = s * _K
    top_w, top_idx = lax.top_k(probs, _K)
    eid = jnp.concatenate([top_idx[:, 0], top_idx[:, 1]]).astype(jnp.int32)
    ohf = (eid[:, None] == jnp.arange(_E, dtype=jnp.int32)[None, :])
    ohf = ohf.astype(jnp.float32)
    cum = jnp.cumsum(ohf, axis=0)
    counts = cum[-1].astype(jnp.int32)
    tiles_e = (counts + _BT - 1) // _BT
    tile_start = jnp.cumsum(tiles_e) - tiles_e
    base_f = (tile_start * _BT).astype(jnp.float32)
    dest = jnp.sum(ohf * (base_f[None, :] + cum - 1.0),
                   axis=1).astype(jnp.int32)
    t_ar = jnp.arange(_TMAX, dtype=jnp.int32)
    tile_eid = jnp.sum((tile_start[None, :] <= t_ar[:, None]).astype(jnp.int32),
                       axis=1) - 1
    tile_valid = (t_ar < jnp.sum(tiles_e)).astype(jnp.int32)
    w2s = jnp.concatenate([top_w[:, 0], top_w[:, 1]]) * _SCALE
    return tile_eid, tile_valid, dest, w2s


def kernel(x, Wg_s, Wu_s, Wd_s, Wg, Wu, Wd, Wr):
    b, s, h = x.shape
    flat = x.reshape(s, h)
    base, probs = _shared_router(flat, Wg_s, Wu_s, Wd_s, Wr)
    tile_eid, tile_valid, dest, w2s = _plan(probs)
    xs = _sc_scatter_rows(flat, dest, _TMAX * _BT, 64)
    ys = _group_mlp(xs, Wg, Wu, Wd, tile_eid, tile_valid)
    picked = _sc_gather_rows(ys, dest, 64)
    return _combine(base, picked, w2s).reshape(b, s, h)


# speedup vs baseline: 4.1086x; 1.1658x over previous
import functools

import jax
import jax.numpy as jnp
from jax import lax
from jax.experimental import pallas as pl
from jax.experimental.pallas import tpu as pltpu
from jax.experimental.pallas import tpu_sc as plsc

_H = 768
_I = 256
_E = 64
_K = 2
_SCALE = 1.0
_BT = 128
_TMAX = 96
_TM = 256


def _shared_router_body(x_ref, wg_ref, wu_ref, wd_ref, wr_ref,
                        base_ref, idx_ref, w_ref):
    xt = x_ref[...]
    g = jnp.dot(xt, wg_ref[...], preferred_element_type=jnp.float32)
    u = jnp.dot(xt, wu_ref[...], preferred_element_type=jnp.float32)
    h = jax.nn.sigmoid(g) * u
    so = jnp.dot(h, wd_ref[...], preferred_element_type=jnp.float32)
    base_ref[...] = xt + so
    logits = jnp.dot(xt, wr_ref[...], preferred_element_type=jnp.float32)
    m = jnp.max(logits, axis=-1, keepdims=True)
    e = jnp.exp(logits - m)
    p = e / jnp.sum(e, axis=-1, keepdims=True)
    i1 = jnp.argmax(p, axis=-1).astype(jnp.int32)
    m1 = jnp.max(p, axis=-1)
    lane = lax.broadcasted_iota(jnp.int32, p.shape, 1)
    p2 = jnp.where(lane == i1[:, None], -1.0, p)
    i2 = jnp.argmax(p2, axis=-1).astype(jnp.int32)
    m2 = jnp.max(p2, axis=-1)
    idx_ref[...] = jnp.stack([i1, i2], axis=0)
    w_ref[...] = jnp.stack([m1, m2], axis=0)


def _shared_router(x2d, Wg_s, Wu_s, Wd_s, Wr):
    s = x2d.shape[0]
    return pl.pallas_call(
        _shared_router_body,
        grid=(s // _TM,),
        in_specs=[
            pl.BlockSpec((_TM, _H), lambda i: (i, 0)),
            pl.BlockSpec((_H, _I), lambda i: (0, 0)),
            pl.BlockSpec((_H, _I), lambda i: (0, 0)),
            pl.BlockSpec((_I, _H), lambda i: (0, 0)),
            pl.BlockSpec((_H, _E), lambda i: (0, 0)),
        ],
        out_specs=[
            pl.BlockSpec((_TM, _H), lambda i: (i, 0)),
            pl.BlockSpec((_K, _TM), lambda i: (0, i)),
            pl.BlockSpec((_K, _TM), lambda i: (0, i)),
        ],
        out_shape=[
            jax.ShapeDtypeStruct((s, _H), jnp.float32),
            jax.ShapeDtypeStruct((_K, s), jnp.int32),
            jax.ShapeDtypeStruct((_K, s), jnp.float32),
        ],
    )(x2d, Wg_s, Wu_s, Wd_s, Wr)


def _group_mlp_body(te_ref, tv_ref, xs_ref, wg_ref, wu_ref, wd_ref, out_ref):
    del te_ref
    t = pl.program_id(0)

    @pl.when(tv_ref[t] == 1)
    def _():
        xt = xs_ref[...].astype(jnp.bfloat16)
        g = jnp.dot(xt, wg_ref[0].astype(jnp.bfloat16),
                    preferred_element_type=jnp.float32)
        u = jnp.dot(xt, wu_ref[0].astype(jnp.bfloat16),
                    preferred_element_type=jnp.float32)
        h = (jax.nn.sigmoid(g) * u).astype(jnp.bfloat16)
        out_ref[...] = jnp.dot(h, wd_ref[0].astype(jnp.bfloat16),
                               preferred_element_type=jnp.float32)


def _group_mlp(xs, Wg, Wu, Wd, tile_eid, tile_valid):
    grid_spec = pltpu.PrefetchScalarGridSpec(
        num_scalar_prefetch=2,
        grid=(_TMAX,),
        in_specs=[
            pl.BlockSpec((_BT, _H), lambda t, te, tv: (t, 0)),
            pl.BlockSpec((1, _H, _I), lambda t, te, tv: (te[t], 0, 0)),
            pl.BlockSpec((1, _H, _I), lambda t, te, tv: (te[t], 0, 0)),
            pl.BlockSpec((1, _I, _H), lambda t, te, tv: (te[t], 0, 0)),
        ],
        out_specs=pl.BlockSpec((_BT, _H), lambda t, te, tv: (t, 0)),
    )
    return pl.pallas_call(
        _group_mlp_body,
        grid_spec=grid_spec,
        out_shape=jax.ShapeDtypeStruct((_TMAX * _BT, _H), jnp.float32),
    )(tile_eid, tile_valid, xs, Wg, Wu, Wd)


def _sc_gather_rows(table, idx, chunk):
    b = idx.shape[0]
    d = table.shape[1]
    nw = 32
    b_per_w = b // nw
    mesh = plsc.VectorSubcoreMesh(core_axis_name="c", subcore_axis_name="s",
                                  num_cores=2, num_subcores=16)

    @functools.partial(
        pl.kernel,
        out_type=jax.ShapeDtypeStruct((b, d), jnp.float32),
        mesh=mesh,
        scratch_types=[
            pltpu.VMEM((chunk,), jnp.int32),
            pltpu.VMEM((chunk, d), jnp.float32),
            pltpu.SemaphoreType.DMA,
        ],
    )
    def k(table_hbm, idx_hbm, out_hbm, idx_v, rows_v, sem):
        wid = lax.axis_index("s") * 2 + lax.axis_index("c")
        base = wid * b_per_w

        @pl.loop(0, b_per_w, step=chunk)
        def _(off):
            pltpu.sync_copy(idx_hbm.at[pl.ds(base + off, chunk)], idx_v)
            pltpu.async_copy(table_hbm.at[idx_v], rows_v, sem).wait()
            pltpu.sync_copy(rows_v, out_hbm.at[pl.ds(base + off, chunk)])

    return k(table, idx)


def _sc_scatter_rows(table, dst_idx, out_rows, chunk):
    s, d = table.shape
    n = dst_idx.shape[0]
    nw = 32
    b_per_w = n // nw
    mesh = plsc.VectorSubcoreMesh(core_axis_name="c", subcore_axis_name="s",
                                  num_cores=2, num_subcores=16)

    @functools.partial(
        pl.kernel,
        out_type=jax.ShapeDtypeStruct((out_rows, d), jnp.float32),
        mesh=mesh,
        scratch_types=[
            pltpu.VMEM((chunk,), jnp.int32),
            pltpu.VMEM((chunk, d), jnp.float32),
            pltpu.SemaphoreType.DMA,
        ],
    )
    def k(table_hbm, idx_hbm, out_hbm, idx_v, rows_v, sem):
        wid = lax.axis_index("s") * 2 + lax.axis_index("c")
        base = wid * b_per_w

        @pl.loop(0, b_per_w, step=chunk)
        def _(off):
            i0 = base + off
            src = lax.rem(i0, s)
            pltpu.sync_copy(idx_hbm.at[pl.ds(i0, chunk)], idx_v)
            pltpu.sync_copy(table_hbm.at[pl.ds(src, chunk)], rows_v)
            pltpu.async_copy(rows_v, out_hbm.at[idx_v], sem).wait()

    return k(table, dst_idx)


def _combine_body(base_ref, p0_ref, p1_ref, w_ref, out_ref):
    w0 = w_ref[0, 0, :][:, None]
    w1 = w_ref[1, 0, :][:, None]
    out_ref[...] = base_ref[...] + w0 * p0_ref[...] + w1 * p1_ref[...]


def _combine(base, picked, w2s):
    s = base.shape[0]
    return pl.pallas_call(
        _combine_body,
        grid=(s // _TM,),
        in_specs=[
            pl.BlockSpec((_TM, _H), lambda i: (i, 0)),
            pl.BlockSpec((_TM, _H), lambda i: (i, 0)),
            pl.BlockSpec((_TM, _H), lambda i, _o=s // _TM: (_o + i, 0)),
            pl.BlockSpec((_K, 1, _TM), lambda i: (0, 0, i)),
        ],
        out_specs=pl.BlockSpec((_TM, _H), lambda i: (i, 0)),
        out_shape=jax.ShapeDtypeStruct((s, _H), jnp.float32),
    )(base, picked, picked, w2s.reshape(_K, 1, s))


def _plan(top_idx2s):
    n = top_idx2s.size
    nb = n // _BT
    eid = top_idx2s.reshape(n)
    ohf = (eid[:, None] == jnp.arange(_E, dtype=jnp.int32)[None, :])
    oh3 = ohf.astype(jnp.float32).reshape(nb, _BT, _E)
    ltri = jnp.tril(jnp.ones((_BT, _BT), jnp.float32), -1)
    intra = jnp.einsum('lm,bme->ble', ltri, oh3)
    btot = jnp.sum(oh3, axis=1)
    boff = jnp.cumsum(btot, axis=0) - btot
    counts = jnp.sum(btot, axis=0).astype(jnp.int32)
    tiles_e = (counts + _BT - 1) // _BT
    tile_start = jnp.cumsum(tiles_e) - tiles_e
    base_f = (tile_start * _BT).astype(jnp.float32)
    rank3 = intra + boff[:, None, :] + base_f[None, None, :]
    dest = jnp.sum(oh3 * rank3, axis=2).reshape(n).astype(jnp.int32)
    t_ar = jnp.arange(_TMAX, dtype=jnp.int32)
    tile_eid = jnp.sum((tile_start[None, :] <= t_ar[:, None]).astype(jnp.int32),
                       axis=1) - 1
    tile_valid = (t_ar < jnp.sum(tiles_e)).astype(jnp.int32)
    return tile_eid, tile_valid, dest


def kernel(x, Wg_s, Wu_s, Wd_s, Wg, Wu, Wd, Wr):
    b, s, h = x.shape
    flat = x.reshape(s, h)
    base, top_idx2s, top_w2s = _shared_router(flat, Wg_s, Wu_s, Wd_s, Wr)
    tile_eid, tile_valid, dest = _plan(top_idx2s)
    xs = _sc_scatter_rows(flat, dest, _TMAX * _BT, 64)
    ys = _group_mlp(xs, Wg, Wu, Wd, tile_eid, tile_valid)
    picked = _sc_gather_rows(ys, dest, 64)
    return _combine(base, picked, top_w2s * _SCALE).reshape(b, s, h)


# speedup vs baseline: 4.5798x; 1.1147x over previous
import functools

import jax
import jax.numpy as jnp
from jax import lax
from jax.experimental import pallas as pl
from jax.experimental.pallas import tpu as pltpu
from jax.experimental.pallas import tpu_sc as plsc

_H = 768
_I = 256
_E = 64
_K = 2
_SCALE = 1.0
_BT = 128
_TMAX = 96
_TM = 256


def _shared_router_body(x_ref, wg_ref, wu_ref, wd_ref, wr_ref,
                        base_ref, idx_ref, w_ref):
    xt = x_ref[...]
    g = jnp.dot(xt, wg_ref[...], preferred_element_type=jnp.float32)
    u = jnp.dot(xt, wu_ref[...], preferred_element_type=jnp.float32)
    h = jax.nn.sigmoid(g) * u
    so = jnp.dot(h, wd_ref[...], preferred_element_type=jnp.float32)
    base_ref[...] = xt + so
    logits = jnp.dot(xt, wr_ref[...], preferred_element_type=jnp.float32)
    m = jnp.max(logits, axis=-1, keepdims=True)
    e = jnp.exp(logits - m)
    p = e / jnp.sum(e, axis=-1, keepdims=True)
    i1 = jnp.argmax(p, axis=-1).astype(jnp.int32)
    m1 = jnp.max(p, axis=-1)
    lane = lax.broadcasted_iota(jnp.int32, p.shape, 1)
    p2 = jnp.where(lane == i1[:, None], -1.0, p)
    i2 = jnp.argmax(p2, axis=-1).astype(jnp.int32)
    m2 = jnp.max(p2, axis=-1)
    idx_ref[...] = jnp.stack([i1, i2], axis=0)
    w_ref[...] = jnp.stack([m1, m2], axis=0)


def _shared_router(x2d, Wg_s, Wu_s, Wd_s, Wr):
    s = x2d.shape[0]
    return pl.pallas_call(
        _shared_router_body,
        grid=(s // _TM,),
        in_specs=[
            pl.BlockSpec((_TM, _H), lambda i: (i, 0)),
            pl.BlockSpec((_H, _I), lambda i: (0, 0)),
            pl.BlockSpec((_H, _I), lambda i: (0, 0)),
            pl.BlockSpec((_I, _H), lambda i: (0, 0)),
            pl.BlockSpec((_H, _E), lambda i: (0, 0)),
        ],
        out_specs=[
            pl.BlockSpec((_TM, _H), lambda i: (i, 0)),
            pl.BlockSpec((_K, _TM), lambda i: (0, i)),
            pl.BlockSpec((_K, _TM), lambda i: (0, i)),
        ],
        out_shape=[
            jax.ShapeDtypeStruct((s, _H), jnp.float32),
            jax.ShapeDtypeStruct((_K, s), jnp.int32),
            jax.ShapeDtypeStruct((_K, s), jnp.float32),
        ],
    )(x2d, Wg_s, Wu_s, Wd_s, Wr)


def _group_mlp_body(te_ref, tv_ref, xs_ref, wg_ref, wu_ref, wd_ref, out_ref):
    del te_ref
    t = pl.program_id(0)

    @pl.when(tv_ref[t] == 1)
    def _():
        xt = xs_ref[...].astype(jnp.bfloat16)
        g = jnp.dot(xt, wg_ref[0].astype(jnp.bfloat16),
                    preferred_element_type=jnp.float32)
        u = jnp.dot(xt, wu_ref[0].astype(jnp.bfloat16),
                    preferred_element_type=jnp.float32)
        h = (jax.nn.sigmoid(g) * u).astype(jnp.bfloat16)
        out_ref[...] = jnp.dot(h, wd_ref[0].astype(jnp.bfloat16),
                               preferred_element_type=jnp.float32)


def _group_mlp(xs, Wg, Wu, Wd, tile_eid, tile_valid):
    grid_spec = pltpu.PrefetchScalarGridSpec(
        num_scalar_prefetch=2,
        grid=(_TMAX,),
        in_specs=[
            pl.BlockSpec((_BT, _H), lambda t, te, tv: (t * tv[t], 0)),
            pl.BlockSpec((1, _H, _I), lambda t, te, tv: (te[t], 0, 0)),
            pl.BlockSpec((1, _H, _I), lambda t, te, tv: (te[t], 0, 0)),
            pl.BlockSpec((1, _I, _H), lambda t, te, tv: (te[t], 0, 0)),
        ],
        out_specs=pl.BlockSpec(
            (_BT, _H),
            lambda t, te, tv: (t * tv[t] + (1 - tv[t]) * _TMAX, 0)),
    )
    return pl.pallas_call(
        _group_mlp_body,
        grid_spec=grid_spec,
        out_shape=jax.ShapeDtypeStruct(((_TMAX + 1) * _BT, _H), jnp.float32),
    )(tile_eid, tile_valid, xs, Wg, Wu, Wd)


def _sc_gather_rows(table, idx, chunk):
    b = idx.shape[0]
    d = table.shape[1]
    nw = 32
    b_per_w = b // nw
    mesh = plsc.VectorSubcoreMesh(core_axis_name="c", subcore_axis_name="s",
                                  num_cores=2, num_subcores=16)

    @functools.partial(
        pl.kernel,
        out_type=jax.ShapeDtypeStruct((b, d), table.dtype),
        mesh=mesh,
        scratch_types=[
            pltpu.VMEM((chunk,), jnp.int32),
            pltpu.VMEM((chunk, d), table.dtype),
            pltpu.SemaphoreType.DMA,
        ],
    )
    def k(table_hbm, idx_hbm, out_hbm, idx_v, rows_v, sem):
        wid = lax.axis_index("s") * 2 + lax.axis_index("c")
        base = wid * b_per_w

        @pl.loop(0, b_per_w, step=chunk)
        def _(off):
            pltpu.sync_copy(idx_hbm.at[pl.ds(base + off, chunk)], idx_v)
            pltpu.async_copy(table_hbm.at[idx_v], rows_v, sem).wait()
            pltpu.sync_copy(rows_v, out_hbm.at[pl.ds(base + off, chunk)])

    return k(table, idx)


def _sc_scatter_rows(table, dst_idx, out_rows, chunk):
    s, d = table.shape
    n = dst_idx.shape[0]
    nw = 32
    b_per_w = n // nw
    mesh = plsc.VectorSubcoreMesh(core_axis_name="c", subcore_axis_name="s",
                                  num_cores=2, num_subcores=16)

    @functools.partial(
        pl.kernel,
        out_type=jax.ShapeDtypeStruct((out_rows, d), table.dtype),
        mesh=mesh,
        scratch_types=[
            pltpu.VMEM((chunk,), jnp.int32),
            pltpu.VMEM((chunk, d), table.dtype),
            pltpu.SemaphoreType.DMA,
        ],
    )
    def k(table_hbm, idx_hbm, out_hbm, idx_v, rows_v, sem):
        wid = lax.axis_index("s") * 2 + lax.axis_index("c")
        base = wid * b_per_w

        @pl.loop(0, b_per_w, step=chunk)
        def _(off):
            i0 = base + off
            src = lax.rem(i0, s)
            pltpu.sync_copy(idx_hbm.at[pl.ds(i0, chunk)], idx_v)
            pltpu.sync_copy(table_hbm.at[pl.ds(src, chunk)], rows_v)
            pltpu.async_copy(rows_v, out_hbm.at[idx_v], sem).wait()

    return k(table, dst_idx)


def _combine_body(base_ref, p0_ref, p1_ref, w_ref, out_ref):
    w0 = w_ref[0, 0, :][:, None]
    w1 = w_ref[1, 0, :][:, None]
    out_ref[...] = (base_ref[...]
                    + w0 * p0_ref[...].astype(jnp.float32)
                    + w1 * p1_ref[...].astype(jnp.float32))


def _combine(base, picked, w2s):
    s = base.shape[0]
    return pl.pallas_call(
        _combine_body,
        grid=(s // _TM,),
        in_specs=[
            pl.BlockSpec((_TM, _H), lambda i: (i, 0)),
            pl.BlockSpec((_TM, _H), lambda i: (i, 0)),
            pl.BlockSpec((_TM, _H), lambda i, _o=s // _TM: (_o + i, 0)),
            pl.BlockSpec((_K, 1, _TM), lambda i: (0, 0, i)),
        ],
        out_specs=pl.BlockSpec((_TM, _H), lambda i: (i, 0)),
        out_shape=jax.ShapeDtypeStruct((s, _H), jnp.float32),
    )(base, picked, picked, w2s.reshape(_K, 1, s))


def _plan(top_idx2s):
    n = top_idx2s.size
    nb = n // _BT
    eid = top_idx2s.reshape(n)
    ohf = (eid[:, None] == jnp.arange(_E, dtype=jnp.int32)[None, :])
    oh3 = ohf.astype(jnp.float32).reshape(nb, _BT, _E)
    ltri = jnp.tril(jnp.ones((_BT, _BT), jnp.float32), -1)
    intra = jnp.einsum('lm,bme->ble', ltri, oh3)
    btot = jnp.sum(oh3, axis=1)
    boff = jnp.cumsum(btot, axis=0) - btot
    counts = jnp.sum(btot, axis=0).astype(jnp.int32)
    tiles_e = (counts + _BT - 1) // _BT
    tile_start = jnp.cumsum(tiles_e) - tiles_e
    base_f = (tile_start * _BT).astype(jnp.float32)
    rank3 = intra + boff[:, None, :] + base_f[None, None, :]
    dest = jnp.sum(oh3 * rank3, axis=2).reshape(n).astype(jnp.int32)
    t_ar = jnp.arange(_TMAX, dtype=jnp.int32)
    tile_eid = jnp.sum((tile_start[None, :] <= t_ar[:, None]).astype(jnp.int32),
                       axis=1) - 1
    tile_valid = (t_ar < jnp.sum(tiles_e)).astype(jnp.int32)
    return tile_eid, tile_valid, dest


def kernel(x, Wg_s, Wu_s, Wd_s, Wg, Wu, Wd, Wr):
    b, s, h = x.shape
    flat = x.reshape(s, h)
    base, top_idx2s, top_w2s = _shared_router(flat, Wg_s, Wu_s, Wd_s, Wr)
    tile_eid, tile_valid, dest = _plan(top_idx2s)
    xs = _sc_scatter_rows(flat, dest, _TMAX * _BT, 64)
    ys = _group_mlp(xs, Wg, Wu, Wd, tile_eid, tile_valid)
    picked = _sc_gather_rows(ys, dest, 64)
    return _combine(base, picked, top_w2s * _SCALE).reshape(b, s, h)
